# baseline (speedup 1.0000x reference)
import functools

import jax
import jax.numpy as jnp
from jax import lax
from jax.experimental import pallas as pl
from jax.experimental.pallas import tpu as pltpu
from jax.experimental.pallas import tpu_sc as plsc

NUM_CONCEPTS = 1000000
DIM = 32
BATCH = 16384
EPS = 1e-23

_NC = 2
_NS = 16
_NW = _NC * _NS
_BPW = BATCH // _NW
_GCH = 128
_NG = _BPW // _GCH


def _sc_gather(idx_i, idx_j, emb):
    mesh = plsc.VectorSubcoreMesh(core_axis_name="c", subcore_axis_name="s")

    @functools.partial(
        pl.kernel,
        mesh=mesh,
        compiler_params=pltpu.CompilerParams(use_tc_tiling_on_sc=False),
        out_type=jax.ShapeDtypeStruct((BATCH, 4 * DIM), jnp.float32),
        scratch_types=[
            pltpu.VMEM((_BPW,), jnp.int32),
            pltpu.VMEM((_BPW,), jnp.int32),
            pltpu.VMEM((_BPW, 2 * DIM), jnp.float32),
            pltpu.VMEM((_BPW, 2 * DIM), jnp.float32),
            pltpu.SemaphoreType.DMA,
        ],
    )
    def gather_kernel(idx_i_hbm, idx_j_hbm, emb_hbm, out_hbm,
                      ii_v, ij_v, ri_v, rj_v, sem):
        wid = lax.axis_index("s") * _NC + lax.axis_index("c")
        base = wid * _BPW
        pltpu.sync_copy(idx_i_hbm.at[pl.ds(base, _BPW)], ii_v)
        pltpu.sync_copy(idx_j_hbm.at[pl.ds(base, _BPW)], ij_v)
        copies = []
        for g in range(_NG):
            sl = pl.ds(g * _GCH, _GCH)
            copies.append(pltpu.async_copy(
                emb_hbm.at[ii_v.at[sl]], ri_v.at[sl], sem))
            copies.append(pltpu.async_copy(
                emb_hbm.at[ij_v.at[sl]], rj_v.at[sl], sem))
        for c in copies:
            c.wait()
        pltpu.sync_copy(ri_v, out_hbm.at[pl.ds(base, _BPW), pl.ds(0, 2 * DIM)])
        pltpu.sync_copy(rj_v, out_hbm.at[pl.ds(base, _BPW), pl.ds(2 * DIM, 2 * DIM)])

    return gather_kernel(idx_i, idx_j, emb)


def _softplus(x):
    return jnp.maximum(x, 0.0) + jnp.log1p(jnp.exp(-jnp.abs(x)))


def _tc_math_body(x_ref, out_ref):
    x = x_ref[...]
    z_i = x[:, 0 * DIM:1 * DIM]
    d_i = x[:, 1 * DIM:2 * DIM]
    z_j = x[:, 2 * DIM:3 * DIM]
    d_j = x[:, 3 * DIM:4 * DIM]
    sp_di = _softplus(d_i)
    sp_dj = _softplus(d_j)
    Z_i = z_i + sp_di
    Z_j = z_j + sp_dj
    side_int = _softplus(jnp.minimum(Z_i, Z_j) - jnp.maximum(z_i, z_j))
    side_j = _softplus(sp_dj)
    ratio = (side_int + EPS) / (side_j + EPS)
    r = ratio
    w = DIM
    while w > 1:
        w //= 2
        r = r[:, :w] * r[:, w:]
    p = r[:, 0]
    out_ref[...] = jnp.clip(p, 1e-7, 1.0 - 1e-7)


def _tc_math(rows):
    blk = 2048
    grid = BATCH // blk
    return pl.pallas_call(
        _tc_math_body,
        grid=(grid,),
        in_specs=[pl.BlockSpec((blk, 4 * DIM), lambda b: (b, 0))],
        out_specs=pl.BlockSpec((blk,), lambda b: (b,)),
        out_shape=jax.ShapeDtypeStruct((BATCH,), jnp.float32),
    )(rows)


def kernel(idx_i, idx_j, emb):
    idx_i = idx_i.astype(jnp.int32)
    idx_j = idx_j.astype(jnp.int32)
    rows = _sc_gather(idx_i, idx_j, emb)
    return _tc_math(rows)
